# baseline (speedup 1.0000x reference)
import functools
import jax
import jax.numpy as jnp
from jax import lax
from jax.experimental import pallas as pl
from jax.experimental.pallas import tpu as pltpu
from jax.experimental.pallas import tpu_sc as plsc

_LOSS_WEIGHT = 1.0

_N, _C, _H, _W = 4, 96, 384, 384
_NW = 32
_C_SC = 24
_C_TC = _C - _C_SC
_PPW = (_N * _C_SC) // _NW
_WPN = _NW // _N
_SLAB = 48
_NSLAB = _H // _SLAB
_NCH = _PPW * _NSLAB
_PB = 8


_S = 16


def _tc_body(p_ref, t_ref, m_ref, out_ref, *, steps_per_n):
    n = pl.program_id(0)
    j = pl.program_id(1)

    @pl.when((n == 0) & (j == 0))
    def _init():
        out_ref[0] = 0.0
        out_ref[1] = 0.0
        out_ref[2] = 0.0

    def strip(s, carry):
        acc_sm, acc_ad = carry
        r0 = s * _S
        m = m_ref[0, pl.ds(r0, _S), :]
        for c in range(_PB):
            p = p_ref[c, pl.ds(r0, _S), :]
            t = t_ref[c, pl.ds(r0, _S), :]
            ad = jnp.abs(p - t)
            adm = ad * m
            cmn = jnp.minimum(adm, m)
            acc_sm = acc_sm + cmn * (adm - 0.5 * cmn)
            acc_ad = acc_ad + adm
        return (acc_sm, acc_ad)

    z = jnp.zeros((_S, _W), jnp.float32)
    acc_sm, acc_ad = lax.fori_loop(0, _H // _S, strip, (z, z))
    out_ref[0] += jnp.sum(acc_sm)
    out_ref[1] += jnp.sum(acc_ad)

    @pl.when(j == 0)
    def _cnt():
        out_ref[2] += jnp.sum(m_ref[0])


def _tc_partials(p3, t3, m3):
    nj = _C_TC // _PB
    return pl.pallas_call(
        functools.partial(_tc_body, steps_per_n=nj),
        grid=(_N, nj),
        in_specs=[
            pl.BlockSpec((_PB, _H, _W), lambda n, j: (n * (_C // _PB) + j, 0, 0)),
            pl.BlockSpec((_PB, _H, _W), lambda n, j: (n * (_C // _PB) + j, 0, 0)),
            pl.BlockSpec((1, _H, _W), lambda n, j: (n, 0, 0)),
        ],
        out_specs=pl.BlockSpec(memory_space=pltpu.SMEM),
        out_shape=jax.ShapeDtypeStruct((3,), jnp.float32),
    )(p3, t3, m3)


def _sc_body(p_hbm, t_hbm, m_hbm, out_hbm,
             pb0, pb1, tb0, tb1, mb, accb,
             ps0, ps1, ts0, ts1):
    wid = lax.axis_index("s") * 2 + lax.axis_index("c")
    n = wid // _WPN
    q0 = n * _C + _C_TC + (wid % _WPN) * _PPW

    pbufs = (pb0, pb1)
    tbufs = (tb0, tb1)
    psems = (ps0, ps1)
    tsems = (ts0, ts1)

    accb[0, :] = jnp.zeros((16,), jnp.float32)
    accb[1, :] = jnp.zeros((16,), jnp.float32)

    def src(k):
        j = k // _PPW
        c = k - j * _PPW
        return (q0 + c, j * _SLAB)

    def issue(k, b):
        pidx, r0 = src(k)
        pltpu.async_copy(p_hbm.at[pidx, pl.ds(r0, _SLAB), :], pbufs[b], psems[b])
        pltpu.async_copy(t_hbm.at[pidx, pl.ds(r0, _SLAB), :], tbufs[b], tsems[b])

    issue(0, 0)
    issue(1, 1)

    def chunk(k, b):
        pidx, r0 = src(k)

        @pl.when(k % _PPW == 0)
        def _mask():
            pltpu.sync_copy(m_hbm.at[n, pl.ds(r0, _SLAB), :], mb)

        pltpu.make_async_copy(p_hbm.at[pidx, pl.ds(r0, _SLAB), :], pbufs[b], psems[b]).wait()
        pltpu.make_async_copy(t_hbm.at[pidx, pl.ds(r0, _SLAB), :], tbufs[b], tsems[b]).wait()

        def row(r, acc):
            acc_sm, acc_ad = acc
            for v in range(_W // 16):
                cs = v * 16
                p = pbufs[b][r, pl.ds(cs, 16)]
                t = tbufs[b][r, pl.ds(cs, 16)]
                m = mb[r, pl.ds(cs, 16)]
                ad = jnp.abs(p - t)
                adm = ad * m
                cmn = jnp.minimum(adm, m)
                acc_sm = acc_sm + cmn * (adm - 0.5 * cmn)
                acc_ad = acc_ad + adm
            return (acc_sm, acc_ad)

        zero = jnp.zeros((16,), jnp.float32)
        acc_sm, acc_ad = lax.fori_loop(0, _SLAB, row, (zero, zero))
        accb[0, :] = accb[0, :] + acc_sm
        accb[1, :] = accb[1, :] + acc_ad

        @pl.when(k + 2 < _NCH)
        def _next():
            issue(k + 2, b)

    def outer(k2, carry):
        chunk(k2 * 2, 0)
        chunk(k2 * 2 + 1, 1)
        return carry

    lax.fori_loop(0, _NCH // 2, outer, 0)

    pltpu.sync_copy(accb, out_hbm.at[wid])


def _sc_partials(p3, t3, m3):
    mesh = plsc.VectorSubcoreMesh(core_axis_name="c", subcore_axis_name="s")
    f = pl.kernel(
        _sc_body,
        mesh=mesh,
        out_type=jax.ShapeDtypeStruct((_NW, 2, 16), jnp.float32),
        scratch_types=[
            pltpu.VMEM((_SLAB, _W), jnp.float32),
            pltpu.VMEM((_SLAB, _W), jnp.float32),
            pltpu.VMEM((_SLAB, _W), jnp.float32),
            pltpu.VMEM((_SLAB, _W), jnp.float32),
            pltpu.VMEM((_SLAB, _W), jnp.float32),
            pltpu.VMEM((2, 16), jnp.float32),
            pltpu.SemaphoreType.DMA,
            pltpu.SemaphoreType.DMA,
            pltpu.SemaphoreType.DMA,
            pltpu.SemaphoreType.DMA,
        ],
    )
    return f(p3, t3, m3)


def kernel(pred, target, front_position):
    N, C, H, W = pred.shape
    p3 = pred.reshape(N * C, H, W)
    t3 = target.reshape(N * C, H, W)
    m3 = front_position.reshape(N, H, W).astype(jnp.float32)

    sc = _sc_partials(p3, t3, m3)
    tc = _tc_partials(p3, t3, m3)

    sm = tc[0] + jnp.sum(sc[:, 0, :])
    ad = tc[1] + jnp.sum(sc[:, 1, :])
    cnt = tc[2] * C
    return (sm / cnt * _LOSS_WEIGHT, ad / cnt)


# speedup vs baseline: 1.0017x; 1.0017x over previous
import functools
import jax
import jax.numpy as jnp
from jax import lax
from jax.experimental import pallas as pl
from jax.experimental.pallas import tpu as pltpu
from jax.experimental.pallas import tpu_sc as plsc

_LOSS_WEIGHT = 1.0

_N, _C, _H, _W = 4, 96, 384, 384
_NW = 32
_C_SC = 8
_C_TC = _C - _C_SC
_PPW = (_N * _C_SC) // _NW
_WPN = _NW // _N
_SLAB = 48
_NSLAB = _H // _SLAB
_NCH = _PPW * _NSLAB
_PB = 8


_S = 16


def _tc_body(p_ref, t_ref, m_ref, out_ref, *, steps_per_n):
    n = pl.program_id(0)
    j = pl.program_id(1)

    @pl.when((n == 0) & (j == 0))
    def _init():
        out_ref[0] = 0.0
        out_ref[1] = 0.0
        out_ref[2] = 0.0

    def strip(s, carry):
        acc_sm, acc_ad = carry
        r0 = s * _S
        m = m_ref[0, pl.ds(r0, _S), :]
        for c in range(_PB):
            p = p_ref[c, pl.ds(r0, _S), :]
            t = t_ref[c, pl.ds(r0, _S), :]
            ad = jnp.abs(p - t)
            adm = ad * m
            cmn = jnp.minimum(adm, m)
            acc_sm = acc_sm + cmn * (adm - 0.5 * cmn)
            acc_ad = acc_ad + adm
        return (acc_sm, acc_ad)

    z = jnp.zeros((_S, _W), jnp.float32)
    acc_sm, acc_ad = lax.fori_loop(0, _H // _S, strip, (z, z))
    out_ref[0] += jnp.sum(acc_sm)
    out_ref[1] += jnp.sum(acc_ad)

    @pl.when(j == 0)
    def _cnt():
        out_ref[2] += jnp.sum(m_ref[0])


def _tc_partials(p3, t3, m3):
    nj = _C_TC // _PB
    return pl.pallas_call(
        functools.partial(_tc_body, steps_per_n=nj),
        grid=(_N, nj),
        in_specs=[
            pl.BlockSpec((_PB, _H, _W), lambda n, j: (n * (_C // _PB) + j, 0, 0)),
            pl.BlockSpec((_PB, _H, _W), lambda n, j: (n * (_C // _PB) + j, 0, 0)),
            pl.BlockSpec((1, _H, _W), lambda n, j: (n, 0, 0)),
        ],
        out_specs=pl.BlockSpec(memory_space=pltpu.SMEM),
        out_shape=jax.ShapeDtypeStruct((3,), jnp.float32),
    )(p3, t3, m3)


def _sc_body(p_hbm, t_hbm, m_hbm, out_hbm,
             pb0, pb1, tb0, tb1, mb, accb,
             ps0, ps1, ts0, ts1):
    wid = lax.axis_index("s") * 2 + lax.axis_index("c")
    n = wid // _WPN
    q0 = n * _C + _C_TC + (wid % _WPN) * _PPW

    pbufs = (pb0, pb1)
    tbufs = (tb0, tb1)
    psems = (ps0, ps1)
    tsems = (ts0, ts1)

    accb[0, :] = jnp.zeros((16,), jnp.float32)
    accb[1, :] = jnp.zeros((16,), jnp.float32)

    def src(k):
        j = k // _PPW
        c = k - j * _PPW
        return (q0 + c, j * _SLAB)

    def issue(k, b):
        pidx, r0 = src(k)
        pltpu.async_copy(p_hbm.at[pidx, pl.ds(r0, _SLAB), :], pbufs[b], psems[b])
        pltpu.async_copy(t_hbm.at[pidx, pl.ds(r0, _SLAB), :], tbufs[b], tsems[b])

    issue(0, 0)
    issue(1, 1)

    def chunk(k, b):
        pidx, r0 = src(k)

        @pl.when(k % _PPW == 0)
        def _mask():
            pltpu.sync_copy(m_hbm.at[n, pl.ds(r0, _SLAB), :], mb)

        pltpu.make_async_copy(p_hbm.at[pidx, pl.ds(r0, _SLAB), :], pbufs[b], psems[b]).wait()
        pltpu.make_async_copy(t_hbm.at[pidx, pl.ds(r0, _SLAB), :], tbufs[b], tsems[b]).wait()

        def row(r, acc):
            acc_sm, acc_ad = acc
            for v in range(_W // 16):
                cs = v * 16
                p = pbufs[b][r, pl.ds(cs, 16)]
                t = tbufs[b][r, pl.ds(cs, 16)]
                m = mb[r, pl.ds(cs, 16)]
                ad = jnp.abs(p - t)
                adm = ad * m
                cmn = jnp.minimum(adm, m)
                acc_sm = acc_sm + cmn * (adm - 0.5 * cmn)
                acc_ad = acc_ad + adm
            return (acc_sm, acc_ad)

        zero = jnp.zeros((16,), jnp.float32)
        acc_sm, acc_ad = lax.fori_loop(0, _SLAB, row, (zero, zero))
        accb[0, :] = accb[0, :] + acc_sm
        accb[1, :] = accb[1, :] + acc_ad

        @pl.when(k + 2 < _NCH)
        def _next():
            issue(k + 2, b)

    def outer(k2, carry):
        chunk(k2 * 2, 0)
        chunk(k2 * 2 + 1, 1)
        return carry

    lax.fori_loop(0, _NCH // 2, outer, 0)

    pltpu.sync_copy(accb, out_hbm.at[wid])


def _sc_partials(p3, t3, m3):
    mesh = plsc.VectorSubcoreMesh(core_axis_name="c", subcore_axis_name="s")
    f = pl.kernel(
        _sc_body,
        mesh=mesh,
        out_type=jax.ShapeDtypeStruct((_NW, 2, 16), jnp.float32),
        scratch_types=[
            pltpu.VMEM((_SLAB, _W), jnp.float32),
            pltpu.VMEM((_SLAB, _W), jnp.float32),
            pltpu.VMEM((_SLAB, _W), jnp.float32),
            pltpu.VMEM((_SLAB, _W), jnp.float32),
            pltpu.VMEM((_SLAB, _W), jnp.float32),
            pltpu.VMEM((2, 16), jnp.float32),
            pltpu.SemaphoreType.DMA,
            pltpu.SemaphoreType.DMA,
            pltpu.SemaphoreType.DMA,
            pltpu.SemaphoreType.DMA,
        ],
    )
    return f(p3, t3, m3)


def kernel(pred, target, front_position):
    N, C, H, W = pred.shape
    p3 = pred.reshape(N * C, H, W)
    t3 = target.reshape(N * C, H, W)
    m3 = front_position.reshape(N, H, W).astype(jnp.float32)

    sc = _sc_partials(p3, t3, m3)
    tc = _tc_partials(p3, t3, m3)

    sm = tc[0] + jnp.sum(sc[:, 0, :])
    ad = tc[1] + jnp.sum(sc[:, 1, :])
    cnt = tc[2] * C
    return (sm / cnt * _LOSS_WEIGHT, ad / cnt)
